# baseline (speedup 1.0000x reference)
import functools

import jax
import jax.numpy as jnp
from jax import lax
from jax.experimental import pallas as pl
from jax.experimental.pallas import tpu as pltpu
from jax.experimental.pallas import tpu_sc as plsc

_B = 4
_H = 32
_N = 512
_NP1 = 513
_RB = 528
_NC = 5
_CR = _NP1 * _NC
_CRP = _CR + 3


def _sc_body(ab2_hbm, e_hbm, tab_hbm, t_hbm, out_hbm, gx, rowb, ab2v, idxv,
             gsem0, gsem1, absem0, absem1, isem, osem):
    cid = lax.axis_index("c")
    sid = lax.axis_index("s")
    wid = sid * 2 + cid

    gsems = (gsem0, gsem1)
    absems = (absem0, absem1)

    pltpu.sync_copy(t_hbm, gx.at[0, 0])
    pltpu.sync_copy(t_hbm, gx.at[1, 0])

    h_lo = lax.iota(jnp.int32, 16)
    h_hi = h_lo + 16

    @pl.when(wid > 0)
    def _():
        pltpu.sync_copy(e_hbm.at[0, wid - 1], idxv.at[0])
        pltpu.async_copy(tab_hbm.at[idxv.at[0]], gx.at[0, pl.ds(1, _N)],
                         gsem0)
        pltpu.async_copy(e_hbm.at[1, wid - 1], idxv.at[1], isem)

    pltpu.async_copy(ab2_hbm.at[0, wid], ab2v.at[0], absem0)

    @pl.loop(wid, _NP1, step=32)
    def _row(io):
        for b in range(_B):
            slot = b & 1
            nslot = 1 - slot
            gxs = gx.at[slot]
            rowbs = rowb.at[slot]
            ab2vs = ab2v.at[slot]

            if b < 3:
                nb, nio = b + 1, io
                gpred = io > 0
                apred = None
            else:
                nb, nio = 0, io + 32
                gpred = nio < _NP1
                apred = gpred
            if b < 2:
                i2b, i2io = b + 2, io
                ipred = io > 0
            else:
                i2b, i2io = (b + 2) % 4, io + 32
                ipred = i2io < _NP1

            @pl.when(gpred)
            def _():
                pltpu.make_async_copy(e_hbm.at[nb, nio - 1],
                                      idxv.at[(b + 1) % 4], isem).wait()
                pltpu.async_copy(tab_hbm.at[idxv.at[(b + 1) % 4]],
                                 gx.at[nslot, pl.ds(1, _N)], gsems[nslot])

            @pl.when(ipred)
            def _():
                pltpu.async_copy(e_hbm.at[i2b, i2io - 1],
                                 idxv.at[(b + 2) % 4], isem)

            def _ab2_prefetch():
                pltpu.async_copy(ab2_hbm.at[nb, nio], ab2v.at[nslot],
                                 absems[nslot])

            if apred is None:
                _ab2_prefetch()
            else:
                pl.when(apred)(_ab2_prefetch)

            not_tok = io > 0

            @pl.when(not_tok)
            def _():
                pltpu.make_async_copy(tab_hbm.at[idxv.at[b]],
                                      gx.at[slot, pl.ds(1, _N)],
                                      gsems[slot]).wait()

            pltpu.make_async_copy(ab2_hbm.at[b, io], ab2vs,
                                  absems[slot]).wait()

            m = not_tok.astype(jnp.int32)

            for c in range(_NC):
                cs = jnp.full((16,), c, jnp.int32)

                @pl.loop(0, 128 if c < _NC - 1 else 1, unroll=8)
                def _col(l):
                    j = c * 128 + l
                    je = j * m
                    glo = gxs[je, pl.ds(0, 16)]
                    ghi = gxs[je, pl.ds(16, 16)]
                    ls = jnp.full((16,), l, jnp.int32)
                    plsc.store_scatter(rowbs, [h_lo, cs, ls], glo)
                    plsc.store_scatter(rowbs, [h_hi, cs, ls], ghi)

            @pl.loop(0, _H)
            def _hadd(h):
                for c in range(_NC):
                    for k in range(8 if c < _NC - 1 else 1):
                        sl = pl.ds(k * 16, 16)
                        abk = ab2vs[pl.ds(c * 128 + k * 16, 16)]
                        rowbs[h, c, sl] = rowbs[h, c, sl] + abk

            def _odrain():
                pltpu.make_async_copy(
                    rowb.at[nslot],
                    out_hbm.at[b, :, pl.ds(io * _NC, _NC), :], osem).wait()

            if b == 0:
                pl.when(io > wid)(_odrain)
            else:
                _odrain()

            pltpu.async_copy(rowbs, out_hbm.at[b, :, pl.ds(io * _NC, _NC), :],
                             osem)

    pltpu.make_async_copy(rowb.at[0], out_hbm.at[0, :, pl.ds(0, _NC), :],
                          osem).wait()


def kernel(attn_bias, edge_input, attn_edge_type, x, edge_encoder_weight,
           graph_token_weight):
    ab2 = attn_bias + attn_bias
    ab2p = jnp.pad(ab2, ((0, 0), (0, 0), (0, _RB - _NP1)))
    e = attn_edge_type.astype(jnp.int32)
    t = graph_token_weight.reshape(_H)

    mesh = plsc.VectorSubcoreMesh(core_axis_name="c", subcore_axis_name="s")
    staged = pl.kernel(
        _sc_body,
        out_type=jax.ShapeDtypeStruct((_B, _H, _CRP, 128), jnp.float32),
        mesh=mesh,
        compiler_params=pltpu.CompilerParams(use_tc_tiling_on_sc=False,
                                             needs_layout_passes=False),
        scratch_types=[
            pltpu.VMEM((2, _RB, _H), jnp.float32),
            pltpu.VMEM((2, _H, _NC, 128), jnp.float32),
            pltpu.VMEM((2, _RB), jnp.float32),
            pltpu.VMEM((4, _N), jnp.int32),
            pltpu.SemaphoreType.DMA,
            pltpu.SemaphoreType.DMA,
            pltpu.SemaphoreType.DMA,
            pltpu.SemaphoreType.DMA,
            pltpu.SemaphoreType.DMA,
            pltpu.SemaphoreType.DMA,
        ],
    )(ab2p, e, edge_encoder_weight, t)

    out = staged[:, :, :_CR, :].reshape(_B, _H, _NP1, _NC * 128)
    return out[:, :, :, :_NP1]


# speedup vs baseline: 1.4585x; 1.4585x over previous
import functools

import jax
import jax.numpy as jnp
from jax import lax
from jax.experimental import pallas as pl
from jax.experimental.pallas import tpu as pltpu
from jax.experimental.pallas import tpu_sc as plsc

_B = 4
_H = 32
_N = 512
_NP1 = 513
_RB = 528
_NC = 5
_CR = _NP1 * _NC
_CRP = _CR + 3


def _sc_body(ab2_hbm, e_hbm, tab_hbm, t_hbm, out_hbm, gx, rowb, ab2v, idxv,
             gsem0, gsem1, absem0, absem1, isem, osem):
    cid = lax.axis_index("c")
    sid = lax.axis_index("s")
    wid = sid * 2 + cid

    gsems = (gsem0, gsem1)
    absems = (absem0, absem1)

    pltpu.sync_copy(t_hbm, gx.at[0, 0])
    pltpu.sync_copy(t_hbm, gx.at[1, 0])

    h_lo = lax.iota(jnp.int32, 16)
    h_hi = h_lo + 16

    @pl.when(wid > 0)
    def _():
        pltpu.sync_copy(e_hbm.at[0, wid - 1], idxv.at[0])
        pltpu.async_copy(tab_hbm.at[idxv.at[0]], gx.at[0, pl.ds(1, _N)],
                         gsem0)
        pltpu.async_copy(e_hbm.at[1, wid - 1], idxv.at[1], isem)

    pltpu.async_copy(ab2_hbm.at[0, wid], ab2v.at[0], absem0)

    @pl.loop(wid, _NP1, step=32)
    def _row(io):
        for b in range(_B):
            slot = b & 1
            nslot = 1 - slot
            gxs = gx.at[slot]
            rowbs = rowb.at[slot]
            ab2vs = ab2v.at[slot]

            if b < 3:
                nb, nio = b + 1, io
                gpred = io > 0
                apred = None
            else:
                nb, nio = 0, io + 32
                gpred = nio < _NP1
                apred = gpred
            if b < 2:
                i2b, i2io = b + 2, io
                ipred = io > 0
            else:
                i2b, i2io = (b + 2) % 4, io + 32
                ipred = i2io < _NP1

            @pl.when(gpred)
            def _():
                pltpu.make_async_copy(e_hbm.at[nb, nio - 1],
                                      idxv.at[(b + 1) % 4], isem).wait()
                pltpu.async_copy(tab_hbm.at[idxv.at[(b + 1) % 4]],
                                 gx.at[nslot, pl.ds(1, _N)], gsems[nslot])

            @pl.when(ipred)
            def _():
                pltpu.async_copy(e_hbm.at[i2b, i2io - 1],
                                 idxv.at[(b + 2) % 4], isem)

            def _ab2_prefetch():
                pltpu.async_copy(ab2_hbm.at[nb, nio], ab2v.at[nslot],
                                 absems[nslot])

            if apred is None:
                _ab2_prefetch()
            else:
                pl.when(apred)(_ab2_prefetch)

            not_tok = io > 0

            @pl.when(not_tok)
            def _():
                pltpu.make_async_copy(tab_hbm.at[idxv.at[b]],
                                      gx.at[slot, pl.ds(1, _N)],
                                      gsems[slot]).wait()

            pltpu.make_async_copy(ab2_hbm.at[b, io], ab2vs,
                                  absems[slot]).wait()

            m = not_tok.astype(jnp.int32)

            for c in range(_NC):
                cs = jnp.full((16,), c, jnp.int32)

                @plsc.parallel_loop(0, 128 if c < _NC - 1 else 1, unroll=8)
                def _col(l):
                    j = c * 128 + l
                    je = j * m
                    glo = gxs[je, pl.ds(0, 16)]
                    ghi = gxs[je, pl.ds(16, 16)]
                    ls = jnp.full((16,), l, jnp.int32)
                    js = ls + c * 128
                    a = plsc.load_gather(ab2vs, [js])
                    plsc.store_scatter(rowbs, [h_lo, cs, ls], glo + a)
                    plsc.store_scatter(rowbs, [h_hi, cs, ls], ghi + a)

            def _odrain():
                pltpu.make_async_copy(
                    rowb.at[nslot],
                    out_hbm.at[b, :, pl.ds(io * _NC, _NC), :], osem).wait()

            if b == 0:
                pl.when(io > wid)(_odrain)
            else:
                _odrain()

            pltpu.async_copy(rowbs, out_hbm.at[b, :, pl.ds(io * _NC, _NC), :],
                             osem)

    pltpu.make_async_copy(rowb.at[0], out_hbm.at[0, :, pl.ds(0, _NC), :],
                          osem).wait()


def kernel(attn_bias, edge_input, attn_edge_type, x, edge_encoder_weight,
           graph_token_weight):
    ab2 = attn_bias + attn_bias
    ab2p = jnp.pad(ab2, ((0, 0), (0, 0), (0, _RB - _NP1)))
    e = attn_edge_type.astype(jnp.int32)
    t = graph_token_weight.reshape(_H)

    mesh = plsc.VectorSubcoreMesh(core_axis_name="c", subcore_axis_name="s")
    staged = pl.kernel(
        _sc_body,
        out_type=jax.ShapeDtypeStruct((_B, _H, _CRP, 128), jnp.float32),
        mesh=mesh,
        compiler_params=pltpu.CompilerParams(use_tc_tiling_on_sc=False,
                                             needs_layout_passes=False),
        scratch_types=[
            pltpu.VMEM((2, _RB, _H), jnp.float32),
            pltpu.VMEM((2, _H, _NC, 128), jnp.float32),
            pltpu.VMEM((2, _RB), jnp.float32),
            pltpu.VMEM((4, _N), jnp.int32),
            pltpu.SemaphoreType.DMA,
            pltpu.SemaphoreType.DMA,
            pltpu.SemaphoreType.DMA,
            pltpu.SemaphoreType.DMA,
            pltpu.SemaphoreType.DMA,
            pltpu.SemaphoreType.DMA,
        ],
    )(ab2p, e, edge_encoder_weight, t)

    out = staged[:, :, :_CR, :].reshape(_B, _H, _NP1, _NC * 128)
    return out[:, :, :, :_NP1]
